# baseline (speedup 1.0000x reference)
import jax
import jax.numpy as jnp
from jax.experimental import pallas as pl
from jax.experimental.pallas import tpu as pltpu

N_TOKENS = 32768
MODEL_DIM = 768
NUM_EXPERTS = 64
BLK = 2048


def _gate_kernel(x_ref, st_ref, gates_ref, mask_ref, temp_ref, out_ref, topk_ref):
    clamp_max = jnp.log(jnp.float32(100.0))
    scale = jnp.exp(jnp.minimum(temp_ref[0, 0], clamp_max))

    st = st_ref[...]
    st_norm = jnp.sqrt(jnp.sum(st * st, axis=1, keepdims=True))
    sn = jnp.transpose(st / jnp.maximum(st_norm, 1e-12))

    x = x_ref[...]
    x_norm = jnp.sqrt(jnp.sum(x * x, axis=1, keepdims=True))
    xn = x / jnp.maximum(x_norm, 1e-12)

    z = jnp.dot(xn, sn, preferred_element_type=jnp.float32)
    cmp = (z * scale > gates_ref[...] * scale) & (mask_ref[...] > 0.0)
    pred_t = jnp.transpose(cmp.astype(jnp.float32))
    out_ref[...] = pred_t
    topk_ref[...] = jnp.sum(pred_t, axis=0).astype(jnp.int32)


def kernel(x, sim_matrix, gates, experts_mask, temperature):
    n_tokens, model_dim = x.shape
    n_experts = sim_matrix.shape[1]
    grid = (n_tokens // BLK,)
    out_t, topk = pl.pallas_call(
        _gate_kernel,
        grid=grid,
        in_specs=[
            pl.BlockSpec((BLK, model_dim), lambda i: (i, 0)),
            pl.BlockSpec((n_experts, model_dim), lambda i: (0, 0)),
            pl.BlockSpec((1, n_experts), lambda i: (0, 0)),
            pl.BlockSpec((1, n_experts), lambda i: (0, 0)),
            pl.BlockSpec((1, 1), lambda i: (0, 0)),
        ],
        out_specs=[
            pl.BlockSpec((n_experts, BLK), lambda i: (0, i)),
            pl.BlockSpec((BLK,), lambda i: (i,)),
        ],
        out_shape=[
            jax.ShapeDtypeStruct((n_experts, n_tokens), jnp.float32),
            jax.ShapeDtypeStruct((n_tokens,), jnp.int32),
        ],
        compiler_params=pltpu.CompilerParams(
            dimension_semantics=("parallel",),
            vmem_limit_bytes=120 * 1024 * 1024,
        ),
    )(x, sim_matrix.T, gates.reshape(1, -1), experts_mask.reshape(1, -1),
      temperature.reshape(1, 1))
    return (out_t.T, topk)


# speedup vs baseline: 1.0201x; 1.0201x over previous
import jax
import jax.numpy as jnp
from jax.experimental import pallas as pl
from jax.experimental.pallas import tpu as pltpu

N_TOKENS = 32768
MODEL_DIM = 768
NUM_EXPERTS = 64
BLK = 8192


def _gate_kernel(x_ref, st_ref, gates_ref, mask_ref, temp_ref, out_ref, topk_ref):
    clamp_max = jnp.log(jnp.float32(100.0))
    scale = jnp.exp(jnp.minimum(temp_ref[0, 0], clamp_max))

    st = st_ref[...]
    st_norm = jnp.sqrt(jnp.sum(st * st, axis=1, keepdims=True))
    sn = jnp.transpose(st / jnp.maximum(st_norm, 1e-12))

    x = x_ref[...]
    x_norm = jnp.sqrt(jnp.sum(x * x, axis=1, keepdims=True))
    xn = x / jnp.maximum(x_norm, 1e-12)

    z = jnp.dot(xn, sn, preferred_element_type=jnp.float32)
    cmp = (z * scale > gates_ref[...] * scale) & (mask_ref[...] > 0.0)
    pred_t = jnp.transpose(cmp.astype(jnp.float32))
    out_ref[...] = pred_t
    topk_ref[...] = jnp.sum(pred_t, axis=0).astype(jnp.int32)


def kernel(x, sim_matrix, gates, experts_mask, temperature):
    n_tokens, model_dim = x.shape
    n_experts = sim_matrix.shape[1]
    grid = (n_tokens // BLK,)
    out_t, topk = pl.pallas_call(
        _gate_kernel,
        grid=grid,
        in_specs=[
            pl.BlockSpec((BLK, model_dim), lambda i: (i, 0)),
            pl.BlockSpec((n_experts, model_dim), lambda i: (0, 0)),
            pl.BlockSpec((1, n_experts), lambda i: (0, 0)),
            pl.BlockSpec((1, n_experts), lambda i: (0, 0)),
            pl.BlockSpec((1, 1), lambda i: (0, 0)),
        ],
        out_specs=[
            pl.BlockSpec((n_experts, BLK), lambda i: (0, i)),
            pl.BlockSpec((BLK,), lambda i: (i,)),
        ],
        out_shape=[
            jax.ShapeDtypeStruct((n_experts, n_tokens), jnp.float32),
            jax.ShapeDtypeStruct((n_tokens,), jnp.int32),
        ],
        compiler_params=pltpu.CompilerParams(
            dimension_semantics=("parallel",),
            vmem_limit_bytes=120 * 1024 * 1024,
        ),
    )(x, sim_matrix.T, gates.reshape(1, -1), experts_mask.reshape(1, -1),
      temperature.reshape(1, 1))
    return (out_t.T, topk)


# speedup vs baseline: 1.0762x; 1.0550x over previous
import jax
import jax.numpy as jnp
from jax.experimental import pallas as pl
from jax.experimental.pallas import tpu as pltpu

N_TOKENS = 32768
MODEL_DIM = 768
NUM_EXPERTS = 64
BLK = 4096


def _gate_kernel(x_ref, st_ref, gates_ref, mask_ref, temp_ref, out_ref, topk_ref):
    clamp_max = jnp.log(jnp.float32(100.0))
    scale = jnp.exp(jnp.minimum(temp_ref[0, 0], clamp_max))

    st = st_ref[...]
    st_norm = jnp.sqrt(jnp.sum(st * st, axis=1, keepdims=True))
    sn = jnp.transpose(st / jnp.maximum(st_norm, 1e-12))

    x = x_ref[...]
    x_norm = jnp.sqrt(jnp.sum(x * x, axis=1, keepdims=True))
    xn = x / jnp.maximum(x_norm, 1e-12)

    z = jnp.dot(xn, sn, preferred_element_type=jnp.float32)
    cmp = (z * scale > gates_ref[...] * scale) & (mask_ref[...] > 0.0)
    pred_t = jnp.transpose(cmp.astype(jnp.float32))
    out_ref[...] = pred_t
    topk_ref[...] = jnp.sum(pred_t, axis=0).astype(jnp.int32)


def kernel(x, sim_matrix, gates, experts_mask, temperature):
    n_tokens, model_dim = x.shape
    n_experts = sim_matrix.shape[1]
    grid = (n_tokens // BLK,)
    out_t, topk = pl.pallas_call(
        _gate_kernel,
        grid=grid,
        in_specs=[
            pl.BlockSpec((BLK, model_dim), lambda i: (i, 0)),
            pl.BlockSpec((n_experts, model_dim), lambda i: (0, 0)),
            pl.BlockSpec((1, n_experts), lambda i: (0, 0)),
            pl.BlockSpec((1, n_experts), lambda i: (0, 0)),
            pl.BlockSpec((1, 1), lambda i: (0, 0)),
        ],
        out_specs=[
            pl.BlockSpec((n_experts, BLK), lambda i: (0, i)),
            pl.BlockSpec((BLK,), lambda i: (i,)),
        ],
        out_shape=[
            jax.ShapeDtypeStruct((n_experts, n_tokens), jnp.float32),
            jax.ShapeDtypeStruct((n_tokens,), jnp.int32),
        ],
        compiler_params=pltpu.CompilerParams(
            dimension_semantics=("parallel",),
            vmem_limit_bytes=120 * 1024 * 1024,
        ),
    )(x, sim_matrix.T, gates.reshape(1, -1), experts_mask.reshape(1, -1),
      temperature.reshape(1, 1))
    return (out_t.T, topk)
